# Initial kernel scaffold; baseline (speedup 1.0000x reference)
import jax
import jax.numpy as jnp
from jax.experimental import pallas as pl


def kernel(node_features, edge_index, W_in, b_in, W1, as1, ad1, b1, W2, as2, ad2, b2, W3, as3, ad3, b3, Wc1, bc1, Wc2, bc2):
    raise NotImplementedError("write your pallas kernel here")


# speedup vs baseline: 43.8100x; 43.8100x over previous
import functools

import jax
import jax.numpy as jnp
from jax import lax
from jax.experimental import pallas as pl
from jax.experimental.pallas import tpu as pltpu
from jax.experimental.pallas import tpu_sc as plsc

N = 10000
E = 320000
M = 10240
HEADS = 8
OC = 16
D = 128
HSW = 144

NW = 32
CHUNK = 128
CHUNKS_PER_W = 79
EDGES_PER_W = CHUNK * CHUNKS_PER_W
E_PAD = NW * EDGES_PER_W
M_ACC = 10016
ROWS_PER_TILE = M_ACC // 16
ROWBLKS = (128, 128, 128, 128, 114)

_f32 = jnp.float32


_R = 1024


def _proj_body(nf, Win, bin_, W1, As, Ad, hs_out, adt_out):
    x0 = jnp.maximum(nf[...] @ Win[...] + bin_[...], 0.0)
    h = x0 @ W1[...]
    hs_out[...] = jnp.concatenate([h, h @ As[...]], axis=1)
    adt_out[...] = h @ Ad[...]


def _combine_proj_body(hs, adt, num2, den2, bias, E16, Wn, As, Ad,
                       hs_out, adt_out):
    h = hs[:, :D]
    z = hs[:, D:] + adt[...]
    wself = jnp.exp(jnp.maximum(z, 0.2 * z))
    den = den2[0] + den2[1] + wself
    num = num2[0] + num2[1] + h * (wself @ E16[...])
    x = num / (den @ E16[...]) + bias[...]
    x = jnp.where(x > 0, x, jnp.exp(jnp.minimum(x, 0.0)) - 1.0)
    hn = x @ Wn[...]
    hs_out[...] = jnp.concatenate([hn, hn @ As[...]], axis=1)
    adt_out[...] = hn @ Ad[...]


def _final_body(hs, adt, num2, den2, E16, Mh, b3, Wc1, bc1, Wc2, bc2, out):
    h = hs[:, :D]
    z = hs[:, D:] + adt[...]
    wself = jnp.exp(jnp.maximum(z, 0.2 * z))
    den = den2[0] + den2[1] + wself
    num = num2[0] + num2[1] + h * (wself @ E16[...])
    x = num / (den @ E16[...])
    x3 = (x @ Mh[...]) * 0.125 + b3[...]
    hc = jnp.maximum(x3 @ Wc1[...] + bc1[...], 0.0)
    out[...] = hc @ Wc2[...] + bc2[...]


def _full(shape):
    return pl.BlockSpec(shape, lambda i: (0,) * len(shape))


def _rows(shape):
    return pl.BlockSpec(shape, lambda i: (i,) + (0,) * (len(shape) - 1))


def _tc_proj(nf, Win, bin_, W1, As, Ad):
    return pl.pallas_call(
        _proj_body,
        grid=(M // _R,),
        in_specs=[_rows((_R, D)), _full((D, D)), _full((1, D)),
                  _full((D, D)), _full((D, 16)), _full((D, 16))],
        out_specs=[_rows((_R, HSW)), _rows((_R, 16))],
        out_shape=[jax.ShapeDtypeStruct((M, HSW), _f32),
                   jax.ShapeDtypeStruct((M, 16), _f32)],
    )(nf, Win, bin_, W1, As, Ad)


def _tc_combine_proj(hs, adt, num2, den2, bias, E16, Wn, As, Ad):
    return pl.pallas_call(
        _combine_proj_body,
        grid=(M // _R,),
        in_specs=[_rows((_R, HSW)), _rows((_R, 16)),
                  pl.BlockSpec((2, _R, D), lambda i: (0, i, 0)),
                  pl.BlockSpec((2, _R, 16), lambda i: (0, i, 0)),
                  _full((1, D)), _full((16, D)), _full((D, D)),
                  _full((D, 16)), _full((D, 16))],
        out_specs=[_rows((_R, HSW)), _rows((_R, 16))],
        out_shape=[jax.ShapeDtypeStruct((M, HSW), _f32),
                   jax.ShapeDtypeStruct((M, 16), _f32)],
    )(hs, adt, num2, den2, bias, E16, Wn, As, Ad)


def _tc_final(hs, adt, num2, den2, E16, Mh, b3, Wc1, bc1, Wc2, bc2):
    return pl.pallas_call(
        _final_body,
        grid=(M // _R,),
        in_specs=[_rows((_R, HSW)), _rows((_R, 16)),
                  pl.BlockSpec((2, _R, D), lambda i: (0, i, 0)),
                  pl.BlockSpec((2, _R, 16), lambda i: (0, i, 0)),
                  _full((16, D)), _full((D, 16)), _full((1, 16)),
                  _full((16, 64)), _full((1, 64)), _full((64, 16)),
                  _full((1, 16))],
        out_specs=[_rows((_R, 16))],
        out_shape=[jax.ShapeDtypeStruct((M, 16), _f32)],
    )(hs, adt, num2, den2, E16, Mh, b3, Wc1, bc1, Wc2, bc2)[0]


_mesh = plsc.VectorSubcoreMesh(core_axis_name="c", subcore_axis_name="s")


@functools.partial(
    pl.kernel,
    out_type=(jax.ShapeDtypeStruct((2, M, D), _f32),
              jax.ShapeDtypeStruct((2, M, 16), _f32)),
    mesh=_mesh,
    scratch_types=(
        pltpu.VMEM((CHUNK,), jnp.int32),
        pltpu.VMEM((CHUNK,), jnp.int32),
        pltpu.VMEM((CHUNK, HSW), _f32),
        pltpu.VMEM((CHUNK, 16), _f32),
        pltpu.VMEM((CHUNK, D), _f32),
        pltpu.VMEM((CHUNK, 16), _f32),
        pltpu.VMEM_SHARED((M_ACC, D), _f32),
        pltpu.VMEM_SHARED((M_ACC, 16), _f32),
        pltpu.SemaphoreType.DMA,
        pltpu.SemaphoreType.DMA,
    ),
    compiler_params=pltpu.CompilerParams(use_tc_tiling_on_sc=False),
)
def _sc_edge(hs_hbm, adt_hbm, src_hbm, dst_hbm, num_hbm, den_hbm,
             s_v, d_v, rows_v, ad_v, msg_v, w_v, num_sh, den_sh, sem1, sem2):
    c = lax.axis_index("c")
    s = lax.axis_index("s")
    wid = c * 16 + s

    zeros16 = jnp.zeros((16,), _f32)

    def zero_msg(i, _):
        msg_v[i // 8, pl.ds((i % 8) * 16, 16)] = zeros16
        return 0

    lax.fori_loop(0, CHUNK * 8, zero_msg, 0)

    def zero_w(i, _):
        w_v[i, :] = zeros16
        return 0

    lax.fori_loop(0, CHUNK, zero_w, 0)

    row0 = s * ROWS_PER_TILE
    off = 0
    for blk in ROWBLKS:
        pltpu.sync_copy(msg_v.at[pl.ds(0, blk)],
                        num_sh.at[pl.ds(row0 + off, blk)])
        pltpu.sync_copy(w_v.at[pl.ds(0, blk)],
                        den_sh.at[pl.ds(row0 + off, blk)])
        off += blk
    plsc.subcore_barrier()

    ebase = wid * EDGES_PER_W

    def chunk_body(g, _):
        base = ebase + g * CHUNK
        pltpu.sync_copy(src_hbm.at[pl.ds(base, CHUNK)], s_v)
        pltpu.sync_copy(dst_hbm.at[pl.ds(base, CHUNK)], d_v)
        cp1 = pltpu.async_copy(hs_hbm.at[s_v], rows_v, sem1)
        cp2 = pltpu.async_copy(adt_hbm.at[d_v], ad_v, sem2)
        cp1.wait()
        cp2.wait()

        def edge_body(e, _):
            z = rows_v[e, pl.ds(D, 16)] + ad_v[e, :]
            w16 = jnp.exp(jnp.maximum(z, 0.2 * z))
            w_v[e, :] = w16
            for j in range(HEADS):
                msg_v[e, pl.ds(j * OC, OC)] = rows_v[e, pl.ds(j * OC, OC)] * w16[j]
            return 0

        lax.fori_loop(0, CHUNK, edge_body, 0)
        pltpu.sync_copy(msg_v, num_sh.at[d_v], add=True)
        pltpu.sync_copy(w_v, den_sh.at[d_v], add=True)
        return 0

    lax.fori_loop(0, CHUNKS_PER_W, chunk_body, 0)
    plsc.subcore_barrier()

    off = 0
    for blk in ROWBLKS:
        r = row0 + off
        pltpu.sync_copy(num_sh.at[pl.ds(r, blk)], num_hbm.at[c, pl.ds(r, blk)])
        pltpu.sync_copy(den_sh.at[pl.ds(r, blk)], den_hbm.at[c, pl.ds(r, blk)])
        off += blk


def _att_mats(a_s, a_d):
    sel = jnp.eye(HEADS, 16, dtype=_f32)
    As = jnp.einsum("hc,hk->hck", a_s, sel).reshape(D, 16)
    Ad = jnp.einsum("hc,hk->hck", a_d, sel).reshape(D, 16)
    return As, Ad


def kernel(node_features, edge_index, W_in, b_in, W1, as1, ad1, b1,
           W2, as2, ad2, b2, W3, as3, ad3, b3, Wc1, bc1, Wc2, bc2):
    nf = jnp.zeros((M, D), _f32).at[:N].set(node_features)
    epad = jnp.full((E_PAD - E,), M_ACC - 1, dtype=jnp.int32)
    src = jnp.concatenate([edge_index[0].astype(jnp.int32), epad])
    dst = jnp.concatenate([edge_index[1].astype(jnp.int32), epad])

    E16 = jnp.concatenate(
        [jnp.kron(jnp.eye(HEADS, dtype=_f32), jnp.ones((1, OC), _f32)),
         jnp.zeros((8, D), _f32)], axis=0)
    Mh = jnp.kron(jnp.ones((HEADS, 1), _f32), jnp.eye(OC, dtype=_f32))

    As1, Ad1 = _att_mats(as1, ad1)
    As2, Ad2 = _att_mats(as2, ad2)
    As3, Ad3 = _att_mats(as3, ad3)

    hs, adt = _tc_proj(nf, W_in, b_in.reshape(1, D), W1, As1, Ad1)
    num2, den2 = _sc_edge(hs, adt, src, dst)
    hs, adt = _tc_combine_proj(hs, adt, num2, den2, b1.reshape(1, D),
                               E16, W2, As2, Ad2)
    num2, den2 = _sc_edge(hs, adt, src, dst)
    hs, adt = _tc_combine_proj(hs, adt, num2, den2, b2.reshape(1, D),
                               E16, W3, As3, Ad3)
    num2, den2 = _sc_edge(hs, adt, src, dst)
    out = _tc_final(hs, adt, num2, den2, E16, Mh, b3.reshape(1, 16),
                    Wc1, bc1.reshape(1, 64), Wc2, bc2.reshape(1, 16))
    return out[:N]
